# Initial kernel scaffold; baseline (speedup 1.0000x reference)
import jax
import jax.numpy as jnp
from jax.experimental import pallas as pl


def kernel(x, W):
    raise NotImplementedError("write your pallas kernel here")


# speedup vs baseline: 1.0016x; 1.0016x over previous
import functools

import jax
import jax.numpy as jnp
from jax import lax
from jax.experimental import pallas as pl

NUM_EXPERTS = 16
TOKENS = 8192
HIDDEN = 2048
TPE = 640
BLK = 1024


def _logits_t_body(w_ref, x_ref, out_ref):
    out_ref[...] = lax.dot_general(
        w_ref[...], x_ref[...],
        (((1,), (1,)), ((), ())),
        preferred_element_type=jnp.float32,
    )


def _logits_t(x, W):
    grid = TOKENS // BLK
    return pl.pallas_call(
        _logits_t_body,
        grid=(grid,),
        in_specs=[
            pl.BlockSpec((NUM_EXPERTS, HIDDEN), lambda i: (0, 0)),
            pl.BlockSpec((BLK, HIDDEN), lambda i: (i, 0)),
        ],
        out_specs=pl.BlockSpec((NUM_EXPERTS, BLK), lambda i: (0, i)),
        out_shape=jax.ShapeDtypeStruct((NUM_EXPERTS, TOKENS), jnp.float32),
    )(W, x)


def kernel(x, W):
    logits_t = _logits_t(x, W)
    _, expert_indices = lax.top_k(logits_t, TPE)
    rows = expert_indices.T.reshape(-1)
    cols = jnp.tile(jnp.arange(NUM_EXPERTS), TPE)
    dispatch_mask = jnp.zeros((TOKENS, NUM_EXPERTS), jnp.float32).at[rows, cols].set(1.0)
    expert_load = dispatch_mask.sum(axis=0)
    loss = (expert_load * jnp.log(expert_load / expert_load.mean())).mean()
    return expert_indices, dispatch_mask, loss


# speedup vs baseline: 1.0692x; 1.0675x over previous
import functools

import jax
import jax.numpy as jnp
from jax import lax
from jax.experimental import pallas as pl
from jax.experimental.pallas import tpu as pltpu
from jax.experimental.pallas import tpu_sc as plsc

NUM_EXPERTS = 16
TOKENS = 8192
HIDDEN = 2048
TPE = 640
BLK = 1024
LANES = 16
ROWS = TOKENS // LANES
NBINS = 256
INT_MIN = -2147483648


def _logits_t_body(w_ref, x_ref, out_ref):
    out_ref[...] = lax.dot_general(
        w_ref[...], x_ref[...],
        (((1,), (1,)), ((), ())),
        preferred_element_type=jnp.float32,
    )


def _logits_t(x, W):
    return pl.pallas_call(
        _logits_t_body,
        grid=(TOKENS // BLK,),
        in_specs=[
            pl.BlockSpec((NUM_EXPERTS, HIDDEN), lambda i: (0, 0)),
            pl.BlockSpec((BLK, HIDDEN), lambda i: (i, 0)),
        ],
        out_specs=pl.BlockSpec((NUM_EXPERTS, BLK), lambda i: (0, i)),
        out_shape=jax.ShapeDtypeStruct((NUM_EXPERTS, TOKENS), jnp.float32),
    )(W, x)


def _row(ref, j):
    return ref[pl.ds(j * LANES, LANES)]


def _sc_body(logits_hbm, idx_out, mask_out,
             lg, ka, kb, ia, ib, hist, maskbuf, allidx):
    core = lax.axis_index("c")
    sub = lax.axis_index("s")
    iota = lax.iota(jnp.int32, LANES)
    ones16 = jnp.ones((LANES,), jnp.float32)

    @pl.when(core == 0)
    def _():
        e = sub
        pltpu.sync_copy(logits_hbm.at[e], lg)

        def prol(j, _):
            src = iota * ROWS + j
            v = plsc.load_gather(lg, [src])
            bits = lax.bitcast_convert_type(v, jnp.int32)
            mkey = jnp.where(bits < 0, jnp.bitwise_not(bits),
                             jnp.bitwise_or(bits, jnp.int32(INT_MIN)))
            skey = jnp.bitwise_xor(mkey, jnp.int32(-1))
            plsc.store_scatter(ka, [j * LANES + iota], skey)
            plsc.store_scatter(ia, [j * LANES + iota], src)
            return 0

        lax.fori_loop(0, ROWS, prol, 0, unroll=4)

        bufs = [(ka, ia, kb, ib), (kb, ib, ka, ia),
                (ka, ia, kb, ib), (kb, ib, None, ia)]
        for p in range(4):
            in_k, in_i, out_k, out_i = bufs[p]
            shift = jnp.int32(8 * p)

            def zero(j, _):
                hist[pl.ds(j * LANES, LANES)] = jnp.zeros((LANES,), jnp.int32)
                return 0

            lax.fori_loop(0, NBINS, zero, 0, unroll=8)

            def count(j, _, in_k=in_k):
                key = _row(in_k, j)
                d = jnp.bitwise_and(
                    lax.shift_right_logical(key, shift), jnp.int32(255))
                addr = d * LANES + iota
                c = plsc.load_gather(hist, [addr])
                plsc.store_scatter(hist, [addr], c + 1)
                return 0

            lax.fori_loop(0, ROWS, count, 0, unroll=4)

            def excl(j, carry):
                v = _row(hist, j)
                inc = jnp.cumsum(v)
                hist[pl.ds(j * LANES, LANES)] = inc - v + carry
                return carry + jnp.sum(v)

            lax.fori_loop(0, NBINS, excl, jnp.zeros((LANES,), jnp.int32),
                          unroll=4)

            def scat(j, _, in_k=in_k, in_i=in_i, out_k=out_k, out_i=out_i,
                     last=(p == 3)):
                key = _row(in_k, j)
                idx = _row(in_i, j)
                d = jnp.bitwise_and(
                    lax.shift_right_logical(key, shift), jnp.int32(255))
                addr = d * LANES + iota
                pos = plsc.load_gather(hist, [addr])
                plsc.store_scatter(hist, [addr], pos + 1)
                if last:
                    plsc.store_scatter(out_i, [pos], idx)
                else:
                    pa = jnp.bitwise_and(pos, jnp.int32(ROWS - 1)) * LANES + \
                        lax.shift_right_logical(pos, jnp.int32(9))
                    plsc.store_scatter(out_k, [pa], key)
                    plsc.store_scatter(out_i, [pa], idx)
                return 0

            lax.fori_loop(0, ROWS, scat, 0, unroll=4)

        pltpu.sync_copy(ia.at[pl.ds(0, TPE)], idx_out.at[e])

    plsc.subcore_barrier()

    @pl.when(core == 0)
    def _():
        w = sub
        base = w * (TOKENS // LANES)
        pltpu.sync_copy(idx_out, allidx)

        def zrow(j, _):
            maskbuf[j] = jnp.zeros((LANES,), jnp.float32)
            return 0

        lax.fori_loop(0, ROWS, zrow, 0, unroll=8)

        for e in range(NUM_EXPERTS):
            ev = jnp.full((LANES,), e, jnp.int32)

            def fill(c, _, e=e, ev=ev):
                idxv = allidx[e, pl.ds(c * LANES, LANES)]
                m = jnp.logical_and(idxv >= base, idxv < base + ROWS)
                plsc.store_scatter(maskbuf, [idxv - base, ev], ones16, mask=m)
                return 0

            lax.fori_loop(0, TPE // LANES, fill, 0, unroll=4)

        pltpu.sync_copy(maskbuf, mask_out.at[pl.ds(base, ROWS)])


def _sc_topk(logits_t):
    mesh = plsc.VectorSubcoreMesh(core_axis_name="c", subcore_axis_name="s")
    f = pl.kernel(
        _sc_body,
        mesh=mesh,
        compiler_params=pltpu.CompilerParams(needs_layout_passes=False),
        out_type=[
            jax.ShapeDtypeStruct((NUM_EXPERTS, TPE), jnp.int32),
            jax.ShapeDtypeStruct((TOKENS, NUM_EXPERTS), jnp.float32),
        ],
        scratch_types=[
            pltpu.VMEM((TOKENS,), jnp.float32),
            pltpu.VMEM((TOKENS,), jnp.int32),
            pltpu.VMEM((TOKENS,), jnp.int32),
            pltpu.VMEM((TOKENS,), jnp.int32),
            pltpu.VMEM((TOKENS,), jnp.int32),
            pltpu.VMEM((NBINS * LANES,), jnp.int32),
            pltpu.VMEM((ROWS, LANES), jnp.float32),
            pltpu.VMEM((NUM_EXPERTS, TPE), jnp.int32),
        ],
    )
    return f(logits_t)


def _loss_body(m_ref, o_ref):
    load = jnp.sum(m_ref[...], axis=0)
    mean = jnp.mean(load)
    o_ref[...] = jnp.reshape(jnp.mean(load * jnp.log(load / mean)), (1, 1))


def _loss(mask):
    out = pl.pallas_call(
        _loss_body,
        out_shape=jax.ShapeDtypeStruct((1, 1), jnp.float32),
    )(mask)
    return jnp.reshape(out, ())


def kernel(x, W):
    logits_t = _logits_t(x, W)
    expert_indices, dispatch_mask, = _sc_topk(logits_t)
    loss = _loss(dispatch_mask)
    return expert_indices, dispatch_mask, loss


# speedup vs baseline: 1.1307x; 1.0575x over previous
import functools

import jax
import jax.numpy as jnp
from jax import lax
from jax.experimental import pallas as pl
from jax.experimental.pallas import tpu as pltpu
from jax.experimental.pallas import tpu_sc as plsc

NUM_EXPERTS = 16
TOKENS = 8192
HIDDEN = 2048
TPE = 640
BLK = 1024
LANES = 16
ROWS = TOKENS // LANES
NBINS = 256
INT_MIN = -2147483648


def _logits_t_body(w_ref, x_ref, out_ref):
    out_ref[...] = lax.dot_general(
        w_ref[...], x_ref[...],
        (((1,), (1,)), ((), ())),
        preferred_element_type=jnp.float32,
    )


def _logits_t(x, W):
    return pl.pallas_call(
        _logits_t_body,
        grid=(TOKENS // BLK,),
        in_specs=[
            pl.BlockSpec((NUM_EXPERTS, HIDDEN), lambda i: (0, 0)),
            pl.BlockSpec((BLK, HIDDEN), lambda i: (i, 0)),
        ],
        out_specs=pl.BlockSpec((NUM_EXPERTS, BLK), lambda i: (0, i)),
        out_shape=jax.ShapeDtypeStruct((NUM_EXPERTS, TOKENS), jnp.float32),
    )(W, x)


def _row(ref, j):
    return ref[pl.ds(j * LANES, LANES)]


def _sc_body(logits_hbm, idx_out, mask_out,
             lg, ka, kb, ia, ib, hist, maskbuf, allidx):
    core = lax.axis_index("c")
    sub = lax.axis_index("s")
    iota = lax.iota(jnp.int32, LANES)
    ones16 = jnp.ones((LANES,), jnp.float32)

    @pl.when(core == 0)
    def _():
        e = sub
        pltpu.sync_copy(logits_hbm.at[e], lg)

        def prol(j, _):
            src = iota * ROWS + j
            v = plsc.load_gather(lg, [src])
            bits = lax.bitcast_convert_type(v, jnp.int32)
            mkey = jnp.where(bits < 0, jnp.bitwise_not(bits),
                             jnp.bitwise_or(bits, jnp.int32(INT_MIN)))
            skey = jnp.bitwise_xor(mkey, jnp.int32(-1))
            ka[pl.ds(j * LANES, LANES)] = skey
            ia[pl.ds(j * LANES, LANES)] = src
            return 0

        lax.fori_loop(0, ROWS, prol, 0, unroll=4)

        bufs = [(ka, ia, kb, ib), (kb, ib, ka, ia),
                (ka, ia, kb, ib), (kb, ib, None, ia)]
        for p in range(4):
            in_k, in_i, out_k, out_i = bufs[p]
            shift = jnp.int32(8 * p)

            def zero(j, _):
                hist[pl.ds(j * LANES, LANES)] = jnp.zeros((LANES,), jnp.int32)
                return 0

            lax.fori_loop(0, NBINS, zero, 0, unroll=8)

            def count(j, _, in_k=in_k):
                key = _row(in_k, j)
                d = jnp.bitwise_and(
                    lax.shift_right_logical(key, shift), jnp.int32(255))
                addr = d * LANES + iota
                plsc.addupdate_scatter(hist, [addr],
                                       jnp.ones((LANES,), jnp.int32))
                return 0

            lax.fori_loop(0, ROWS, count, 0, unroll=4)

            def excl(j, carry):
                v = _row(hist, j)
                inc = jnp.cumsum(v)
                hist[pl.ds(j * LANES, LANES)] = inc - v + carry
                return carry + jnp.sum(v)

            lax.fori_loop(0, NBINS, excl, jnp.zeros((LANES,), jnp.int32),
                          unroll=4)

            def scat(j, _, in_k=in_k, in_i=in_i, out_k=out_k, out_i=out_i,
                     last=(p == 3)):
                key = _row(in_k, j)
                idx = _row(in_i, j)
                d = jnp.bitwise_and(
                    lax.shift_right_logical(key, shift), jnp.int32(255))
                addr = d * LANES + iota
                pos = plsc.load_gather(hist, [addr])
                plsc.store_scatter(hist, [addr], pos + 1)
                if last:
                    plsc.store_scatter(out_i, [pos], idx)
                else:
                    pa = jnp.bitwise_and(pos, jnp.int32(ROWS - 1)) * LANES + \
                        lax.shift_right_logical(pos, jnp.int32(9))
                    plsc.store_scatter(out_k, [pa], key)
                    plsc.store_scatter(out_i, [pa], idx)
                return 0

            lax.fori_loop(0, ROWS, scat, 0, unroll=4)

        pltpu.sync_copy(ia.at[pl.ds(0, TPE)], idx_out.at[e])

    plsc.subcore_barrier()

    @pl.when(core == 0)
    def _():
        w = sub
        base = w * (TOKENS // LANES)
        pltpu.sync_copy(idx_out, allidx)

        def zrow(j, _):
            maskbuf[j] = jnp.zeros((LANES,), jnp.float32)
            return 0

        lax.fori_loop(0, ROWS, zrow, 0, unroll=8)

        for e in range(NUM_EXPERTS):
            ev = jnp.full((LANES,), e, jnp.int32)

            def fill(c, _, e=e, ev=ev):
                idxv = allidx[e, pl.ds(c * LANES, LANES)]
                m = jnp.logical_and(idxv >= base, idxv < base + ROWS)
                plsc.store_scatter(maskbuf, [idxv - base, ev], ones16, mask=m)
                return 0

            lax.fori_loop(0, TPE // LANES, fill, 0, unroll=4)

        pltpu.sync_copy(maskbuf, mask_out.at[pl.ds(base, ROWS)])


def _sc_topk(logits_t):
    mesh = plsc.VectorSubcoreMesh(core_axis_name="c", subcore_axis_name="s")
    f = pl.kernel(
        _sc_body,
        mesh=mesh,
        compiler_params=pltpu.CompilerParams(needs_layout_passes=False),
        out_type=[
            jax.ShapeDtypeStruct((NUM_EXPERTS, TPE), jnp.int32),
            jax.ShapeDtypeStruct((TOKENS, NUM_EXPERTS), jnp.float32),
        ],
        scratch_types=[
            pltpu.VMEM((TOKENS,), jnp.float32),
            pltpu.VMEM((TOKENS,), jnp.int32),
            pltpu.VMEM((TOKENS,), jnp.int32),
            pltpu.VMEM((TOKENS,), jnp.int32),
            pltpu.VMEM((TOKENS,), jnp.int32),
            pltpu.VMEM((NBINS * LANES,), jnp.int32),
            pltpu.VMEM((ROWS, LANES), jnp.float32),
            pltpu.VMEM((NUM_EXPERTS, TPE), jnp.int32),
        ],
    )
    return f(logits_t)


def _loss_body(m_ref, o_ref):
    load = jnp.sum(m_ref[...], axis=0)
    mean = jnp.mean(load)
    o_ref[...] = jnp.reshape(jnp.mean(load * jnp.log(load / mean)), (1, 1))


def _loss(mask):
    out = pl.pallas_call(
        _loss_body,
        out_shape=jax.ShapeDtypeStruct((1, 1), jnp.float32),
    )(mask)
    return jnp.reshape(out, ())


def kernel(x, W):
    logits_t = _logits_t(x, W)
    expert_indices, dispatch_mask, = _sc_topk(logits_t)
    loss = _loss(dispatch_mask)
    return expert_indices, dispatch_mask, loss


# speedup vs baseline: 1.4295x; 1.2642x over previous
import functools

import jax
import jax.numpy as jnp
from jax import lax
from jax.experimental import pallas as pl
from jax.experimental.pallas import tpu as pltpu
from jax.experimental.pallas import tpu_sc as plsc

NUM_EXPERTS = 16
TOKENS = 8192
HIDDEN = 2048
TPE = 640
BLK = 1024
LANES = 16
ROWS = TOKENS // LANES
NBINS = 256
INT_MIN = -2147483648


def _logits_t_body(w_ref, x_ref, out_ref):
    out_ref[...] = lax.dot_general(
        w_ref[...], x_ref[...],
        (((1,), (1,)), ((), ())),
        preferred_element_type=jnp.float32,
    )


def _logits_t(x, W):
    return pl.pallas_call(
        _logits_t_body,
        grid=(TOKENS // BLK,),
        in_specs=[
            pl.BlockSpec((NUM_EXPERTS, HIDDEN), lambda i: (0, 0)),
            pl.BlockSpec((BLK, HIDDEN), lambda i: (i, 0)),
        ],
        out_specs=pl.BlockSpec((NUM_EXPERTS, BLK), lambda i: (0, i)),
        out_shape=jax.ShapeDtypeStruct((NUM_EXPERTS, TOKENS), jnp.float32),
    )(W, x)


def _row(ref, j):
    return ref[pl.ds(j * LANES, LANES)]


def _sc_body(logits_hbm, idx_out, mask_out,
             lg, ka, kb, ia, ib, pb, hist, maskbuf, allidx):
    core = lax.axis_index("c")
    sub = lax.axis_index("s")
    iota = lax.iota(jnp.int32, LANES)
    ones16 = jnp.ones((LANES,), jnp.float32)

    @pl.when(core == 0)
    def _():
        e = sub
        pltpu.sync_copy(logits_hbm.at[e], lg)

        @plsc.parallel_loop(0, ROWS, unroll=4)
        def _prol(j):
            src = iota * ROWS + j
            v = plsc.load_gather(lg, [src])
            bits = lax.bitcast_convert_type(v, jnp.int32)
            mkey = jnp.where(bits < 0, jnp.bitwise_not(bits),
                             jnp.bitwise_or(bits, jnp.int32(INT_MIN)))
            skey = jnp.bitwise_xor(mkey, jnp.int32(-1))
            ka[pl.ds(j * LANES, LANES)] = skey
            ia[pl.ds(j * LANES, LANES)] = src

        bufs = [(ka, ia, kb, ib), (kb, ib, ka, ia),
                (ka, ia, kb, ib), (kb, ib, None, ia)]
        for p in range(4):
            in_k, in_i, out_k, out_i = bufs[p]
            shift = jnp.int32(8 * p)

            @plsc.parallel_loop(0, NBINS, unroll=8)
            def _zero(j):
                hist[pl.ds(j * LANES, LANES)] = jnp.zeros((LANES,), jnp.int32)

            @plsc.parallel_loop(0, ROWS, unroll=4)
            def _count(j, in_k=in_k):
                key = _row(in_k, j)
                d = jnp.bitwise_and(
                    lax.shift_right_logical(key, shift), jnp.int32(255))
                addr = d * LANES + iota
                plsc.addupdate_scatter(hist, [addr],
                                       jnp.ones((LANES,), jnp.int32))

            def excl(j, carry):
                v = _row(hist, j)
                inc = jnp.cumsum(v)
                hist[pl.ds(j * LANES, LANES)] = inc - v + carry
                return carry + jnp.sum(v)

            lax.fori_loop(0, NBINS, excl, jnp.zeros((LANES,), jnp.int32),
                          unroll=4)

            def posl(j, _, in_k=in_k):
                key = _row(in_k, j)
                d = jnp.bitwise_and(
                    lax.shift_right_logical(key, shift), jnp.int32(255))
                addr = d * LANES + iota
                pos = plsc.load_gather(hist, [addr])
                plsc.store_scatter(hist, [addr], pos + 1)
                pb[pl.ds(j * LANES, LANES)] = pos
                return 0

            lax.fori_loop(0, ROWS, posl, 0, unroll=4)

            @plsc.parallel_loop(0, ROWS, unroll=4)
            def _out(j, in_k=in_k, in_i=in_i, out_k=out_k, out_i=out_i,
                     last=(p == 3)):
                idx = _row(in_i, j)
                pos = _row(pb, j)
                if last:
                    plsc.store_scatter(out_i, [pos], idx)
                else:
                    key = _row(in_k, j)
                    pa = jnp.bitwise_and(pos, jnp.int32(ROWS - 1)) * LANES + \
                        lax.shift_right_logical(pos, jnp.int32(9))
                    plsc.store_scatter(out_k, [pa], key)
                    plsc.store_scatter(out_i, [pa], idx)

        pltpu.sync_copy(ia.at[pl.ds(0, TPE)], idx_out.at[e])

    plsc.subcore_barrier()

    @pl.when(core == 0)
    def _():
        w = sub
        base = w * (TOKENS // LANES)
        pltpu.sync_copy(idx_out, allidx)

        @plsc.parallel_loop(0, ROWS, unroll=8)
        def _zrow(j):
            maskbuf[j] = jnp.zeros((LANES,), jnp.float32)

        for e in range(NUM_EXPERTS):
            ev = jnp.full((LANES,), e, jnp.int32)

            @plsc.parallel_loop(0, TPE // LANES, unroll=4)
            def _fill(c, e=e, ev=ev):
                idxv = allidx[e, pl.ds(c * LANES, LANES)]
                m = jnp.logical_and(idxv >= base, idxv < base + ROWS)
                plsc.store_scatter(maskbuf, [idxv - base, ev], ones16, mask=m)

        pltpu.sync_copy(maskbuf, mask_out.at[pl.ds(base, ROWS)])


def _sc_topk(logits_t):
    mesh = plsc.VectorSubcoreMesh(core_axis_name="c", subcore_axis_name="s")
    f = pl.kernel(
        _sc_body,
        mesh=mesh,
        compiler_params=pltpu.CompilerParams(needs_layout_passes=False),
        out_type=[
            jax.ShapeDtypeStruct((NUM_EXPERTS, TPE), jnp.int32),
            jax.ShapeDtypeStruct((TOKENS, NUM_EXPERTS), jnp.float32),
        ],
        scratch_types=[
            pltpu.VMEM((TOKENS,), jnp.float32),
            pltpu.VMEM((TOKENS,), jnp.int32),
            pltpu.VMEM((TOKENS,), jnp.int32),
            pltpu.VMEM((TOKENS,), jnp.int32),
            pltpu.VMEM((TOKENS,), jnp.int32),
            pltpu.VMEM((TOKENS,), jnp.int32),
            pltpu.VMEM((NBINS * LANES,), jnp.int32),
            pltpu.VMEM((ROWS, LANES), jnp.float32),
            pltpu.VMEM((NUM_EXPERTS, TPE), jnp.int32),
        ],
    )
    return f(logits_t)


def _loss_body(m_ref, o_ref):
    load = jnp.sum(m_ref[...], axis=0)
    mean = jnp.mean(load)
    o_ref[...] = jnp.reshape(jnp.mean(load * jnp.log(load / mean)), (1, 1))


def _loss(mask):
    out = pl.pallas_call(
        _loss_body,
        out_shape=jax.ShapeDtypeStruct((1, 1), jnp.float32),
    )(mask)
    return jnp.reshape(out, ())


def kernel(x, W):
    logits_t = _logits_t(x, W)
    expert_indices, dispatch_mask, = _sc_topk(logits_t)
    loss = _loss(dispatch_mask)
    return expert_indices, dispatch_mask, loss
